# baseline (speedup 1.0000x reference)
import functools

import jax
import jax.numpy as jnp
from jax import lax
from jax.experimental import pallas as pl
from jax.experimental.pallas import tpu as pltpu
from jax.experimental.pallas import tpu_sc as plsc

N = 10000
E = 320000
D = 128
NB = 8
KH = 3

NC = 2
NS = 16
NW = NC * NS

NPAD = 10240
RPT = NPAD // NS
ECH = 64
EW = 10240
NCHK = EW // ECH
QCHK = NCHK // 4
ZR = 128
NBUF = 4
EPAD = NW * EW
CW = 128

@functools.cache
def _sc_kernels():
    mesh = plsc.VectorSubcoreMesh(
        core_axis_name="c", subcore_axis_name="s", num_cores=NC, num_subcores=NS
    )
    degree = functools.partial(
        pl.kernel,
        out_type=jax.ShapeDtypeStruct((NC, NPAD, CW), jnp.float32),
        mesh=mesh,
        scratch_types=[
            pltpu.VMEM((NCHK, ECH), jnp.int32),
            pltpu.VMEM((ECH, CW), jnp.float32),
            pltpu.VMEM((ZR, CW), jnp.float32),
            pltpu.VMEM_SHARED((NPAD, CW), jnp.float32),
        ],
    )(_sc_degree_body)
    hop = functools.partial(
        pl.kernel,
        out_type=jax.ShapeDtypeStruct((NC, NPAD, D), jnp.float32),
        mesh=mesh,
        scratch_types=[
            pltpu.VMEM((QCHK, ECH), jnp.int32),
            pltpu.VMEM((QCHK, ECH), jnp.int32),
            pltpu.VMEM((NBUF, ECH, D), jnp.float32),
            pltpu.VMEM_SHARED((NPAD, D), jnp.float32),
            pltpu.SemaphoreType.DMA((NBUF,)),
            pltpu.SemaphoreType.DMA((NBUF,)),
        ],
    )(_sc_hop_body)
    return degree, hop


def _sc_degree_body(src_hbm, cnt_hbm, srcv, onesv, zerov, acc):
    c = lax.axis_index("c")
    s = lax.axis_index("s")
    wid = c * NS + s

    def fill_ones(t, _):
        i = t // (CW // 16)
        k = t % (CW // 16)
        onesv[i, pl.ds(k * 16, 16)] = jnp.full((16,), 1.0, jnp.float32)
        return 0

    def fill_zero(t, _):
        i = t // (CW // 16)
        k = t % (CW // 16)
        zerov[i, pl.ds(k * 16, 16)] = jnp.zeros((16,), jnp.float32)
        return 0

    lax.fori_loop(0, ZR * (CW // 16), fill_zero, 0)
    for r in range(RPT // ZR):
        pltpu.sync_copy(zerov, acc.at[pl.ds(s * RPT + r * ZR, ZR)])
    lax.fori_loop(0, ECH * (CW // 16), fill_ones, 0)
    plsc.subcore_barrier()

    pltpu.sync_copy(src_hbm.at[wid], srcv)

    def step(j, _):
        pltpu.sync_copy(onesv, acc.at[srcv.at[j]], add=True)
        return 0

    lax.fori_loop(0, NCHK, step, 0)
    plsc.subcore_barrier()

    pltpu.sync_copy(acc.at[pl.ds(s * RPT, RPT)],
                    cnt_hbm.at[c, pl.ds(s * RPT, RPT)])


def _sc_hop_body(hs_hbm, src_hbm, dst_hbm, par_hbm, srcv, dstv, rows, acc,
                 gsem, ssem):
    c = lax.axis_index("c")
    s = lax.axis_index("s")
    wid = c * NS + s

    def fill_zero(t, _):
        i = t // (D // 16)
        k = t % (D // 16)
        rows[0, i, pl.ds(k * 16, 16)] = jnp.zeros((16,), jnp.float32)
        return 0

    lax.fori_loop(0, ECH * (D // 16), fill_zero, 0)

    for r in range(RPT // ECH):
        pltpu.sync_copy(rows.at[0], acc.at[pl.ds(s * RPT + r * ECH, ECH)])

    for h in range(4):
        pltpu.sync_copy(src_hbm.at[wid, pl.ds(h * QCHK, QCHK)], srcv)
        pltpu.sync_copy(dst_hbm.at[wid, pl.ds(h * QCHK, QCHK)], dstv)
        if h == 0:
            plsc.subcore_barrier()
        for k in range(NBUF - 1):
            pltpu.async_copy(hs_hbm.at[srcv.at[k]], rows.at[k], gsem.at[k])

        def step(j, _):
            b = j % NBUF
            pltpu.make_async_copy(hs_hbm.at[srcv.at[j]], rows.at[b],
                                  gsem.at[b]).wait()
            pltpu.async_copy(rows.at[b], acc.at[dstv.at[j]], ssem.at[b],
                             add=True)

            nj = j + NBUF - 1
            nb = nj % NBUF

            @pl.when(nj < QCHK)
            def _():
                @pl.when(j > 0)
                def _():
                    pltpu.make_async_copy(rows.at[nb], acc.at[dstv.at[j - 1]],
                                          ssem.at[nb]).wait()

                pltpu.async_copy(hs_hbm.at[srcv.at[nj]], rows.at[nb],
                                 gsem.at[nb])

            return 0

        lax.fori_loop(0, QCHK, step, 0)

        for k in range(NBUF):
            j = QCHK - 1 - k
            pltpu.make_async_copy(rows.at[j % NBUF], acc.at[dstv.at[j]],
                                  ssem.at[j % NBUF]).wait()

    plsc.subcore_barrier()

    pltpu.sync_copy(acc.at[pl.ds(s * RPT, RPT)],
                    par_hbm.at[c, pl.ds(s * RPT, RPT)])


BLK = 256
_GRID = NPAD // BLK


def _kan_body(first, *refs):
    if first:
        (g_ref, w_ref, b_ref, out_ref) = refs
        acc = b_ref[...] + jnp.zeros((BLK, D), jnp.float32)
    else:
        (g_ref, w_ref, b_ref, accin_ref, out_ref) = refs
        acc = accin_ref[...] + b_ref[...]

    g = g_ref[...]
    mu = jnp.mean(g, axis=1, keepdims=True)
    var = jnp.mean((g - mu) ** 2, axis=1, keepdims=True)
    hn = (g - mu) * lax.rsqrt(var + 1e-5)
    scale = (NB - 1) / 4.0
    for j in range(NB):
        gj = -2.0 + j * (4.0 / (NB - 1))
        r = jnp.exp(-(((hn - gj) * scale) ** 2))
        acc = acc + jnp.dot(r, w_ref[j], preferred_element_type=jnp.float32)
    out_ref[...] = acc


_w_spec = pl.BlockSpec((NB, D, D), lambda i: (0, 0, 0))
_b_spec = pl.BlockSpec((1, D), lambda i: (0, 0))
_row_spec = pl.BlockSpec((BLK, D), lambda i: (i, 0))
_pair_spec = pl.BlockSpec((NC, BLK, D), lambda i: (0, i, 0))
_cnt_spec = pl.BlockSpec((NC, BLK, CW), lambda i: (0, i, 0))
_row_shape = jax.ShapeDtypeStruct((NPAD, D), jnp.float32)


def _make_kan(first):
    body = functools.partial(_kan_body, first)
    if first:
        in_specs = [_row_spec, _w_spec, _b_spec]
        aliases = {}
    else:
        in_specs = [_row_spec, _w_spec, _b_spec, _row_spec]
        aliases = {3: 0}
    return pl.pallas_call(
        body,
        grid=(_GRID,),
        in_specs=in_specs,
        out_specs=_row_spec,
        out_shape=_row_shape,
        input_output_aliases=aliases,
    )


_kan_first = _make_kan(True)
_kan_next = _make_kan(False)


def _prescale_body(x_ref, cnt_ref, xs_ref):
    cnt = cnt_ref[0] + cnt_ref[1]
    inv = 1.0 / jnp.maximum(cnt, 1.0)
    xs_ref[...] = x_ref[...] * inv


_prescale = pl.pallas_call(
    _prescale_body,
    grid=(_GRID,),
    in_specs=[_row_spec, _cnt_spec],
    out_specs=_row_spec,
    out_shape=_row_shape,
)


def _combine_body(with_scale, *refs):
    if with_scale:
        (p_ref, cnt_ref, g_ref, gs_ref) = refs
    else:
        (p_ref, g_ref) = refs
    g = p_ref[0] + p_ref[1]
    g_ref[...] = g
    if with_scale:
        cnt = cnt_ref[0] + cnt_ref[1]
        inv = 1.0 / jnp.maximum(cnt, 1.0)
        gs_ref[...] = g * inv


_combine_scaled = pl.pallas_call(
    functools.partial(_combine_body, True),
    grid=(_GRID,),
    in_specs=[_pair_spec, _cnt_spec],
    out_specs=[_row_spec, _row_spec],
    out_shape=[_row_shape, _row_shape],
)

_combine_plain = pl.pallas_call(
    functools.partial(_combine_body, False),
    grid=(_GRID,),
    in_specs=[_pair_spec],
    out_specs=_row_spec,
    out_shape=_row_shape,
)


def kernel(x, edge_index, kan_W, kan_b, hop_weights):
    src = edge_index[0]
    dst = edge_index[1]
    pad = jnp.full((EPAD - E,), N, jnp.int32)
    src_p = jnp.concatenate([src, pad]).reshape(NW, NCHK, ECH)
    dst_p = jnp.concatenate([dst, pad]).reshape(NW, NCHK, ECH)
    xp = jnp.pad(x, ((0, NPAD - N), (0, 0)))

    w = jax.nn.softmax(hop_weights)
    Wp = kan_W.reshape(KH + 1, D, NB, D).transpose(0, 2, 1, 3)
    Wp = Wp * w[:, None, None, None]
    bp = (kan_b * w[:, None])[:, None, :]

    sc_degree, sc_hop = _sc_kernels()
    cnt = sc_degree(src_p)
    xs = _prescale(xp, cnt)
    p1 = sc_hop(xs, src_p, dst_p)
    g1, g1s = _combine_scaled(p1, cnt)
    p2 = sc_hop(g1s, src_p, dst_p)
    g2, g2s = _combine_scaled(p2, cnt)
    p3 = sc_hop(g2s, src_p, dst_p)
    g3 = _combine_plain(p3)
    out = _kan_first(xp, Wp[0], bp[0])
    out = _kan_next(g1, Wp[1], bp[1], out)
    out = _kan_next(g2, Wp[2], bp[2], out)
    out = _kan_next(g3, Wp[3], bp[3], out)
    return out[:N]
